# baseline (speedup 1.0000x reference)
import functools

import jax
import jax.numpy as jnp
from jax import lax
from jax.experimental import pallas as pl
from jax.experimental.pallas import tpu as pltpu
from jax.experimental.pallas import tpu_sc as plsc

BATCH = 16384
EMB = 64
FEAT = 128
HID = 64

NC = 2
NS = 16
NW = NC * NS
B_PER_W = BATCH // NW
CHUNK = 128
NCHUNK = B_PER_W // CHUNK


def _gather_body(table_hbm, idx_hbm, out_hbm, idx_v, rows_v, sem):
    wid = lax.axis_index("s") * NC + lax.axis_index("c")
    base = wid * B_PER_W
    pltpu.sync_copy(idx_hbm.at[wid], idx_v)
    copies = []
    for j in range(NCHUNK):
        copies.append(
            pltpu.async_copy(
                table_hbm.at[idx_v.at[j]],
                rows_v.at[pl.ds(j * CHUNK, CHUNK)],
                sem,
            )
        )
    for c in copies:
        c.wait()
    pltpu.sync_copy(rows_v, out_hbm.at[pl.ds(base, B_PER_W)])


_sc_gather = pl.kernel(
    _gather_body,
    mesh=plsc.VectorSubcoreMesh(core_axis_name="c", subcore_axis_name="s"),
    out_type=jax.ShapeDtypeStruct((BATCH, EMB), jnp.float32),
    scratch_types=[
        pltpu.VMEM((NCHUNK, CHUNK), jnp.int32),
        pltpu.VMEM((B_PER_W, EMB), jnp.float32),
        pltpu.SemaphoreType.DMA,
    ],
    compiler_params=pltpu.CompilerParams(use_tc_tiling_on_sc=False),
)


BM = 1024


def _proj_body(feat_ref, w_ref, b_ref, gath_ref, out_ref):
    left = lax.dot_general(
        feat_ref[...],
        w_ref[...],
        (((1,), (1,)), ((), ())),
        preferred_element_type=jnp.float32,
    )
    left = left + b_ref[...]
    out_ref[...] = jnp.concatenate([left, gath_ref[...]], axis=-1)


def _projector(feat, W, b2, gathered):
    return pl.pallas_call(
        _proj_body,
        grid=(BATCH // BM,),
        in_specs=[
            pl.BlockSpec((BM, FEAT), lambda i: (i, 0)),
            pl.BlockSpec((HID, FEAT), lambda i: (0, 0)),
            pl.BlockSpec((1, HID), lambda i: (0, 0)),
            pl.BlockSpec((BM, EMB), lambda i: (i, 0)),
        ],
        out_specs=pl.BlockSpec((BM, HID + EMB), lambda i: (i, 0)),
        out_shape=jax.ShapeDtypeStruct((BATCH, HID + EMB), jnp.float32),
    )(feat, W, b2, gathered)


def kernel(feat, id, W, b, table):
    ids = id.astype(jnp.int32).reshape(NW, NCHUNK, CHUNK)
    gathered = _sc_gather(table, ids)
    return _projector(feat, W, b.reshape(1, HID), gathered)


# speedup vs baseline: 1.0017x; 1.0017x over previous
import jax
import jax.numpy as jnp
from jax import lax
from jax.experimental import pallas as pl
from jax.experimental.pallas import tpu as pltpu
from jax.experimental.pallas import tpu_sc as plsc

BATCH = 16384
EMB = 64
FEAT = 128
HID = 64

NC = 2
NS = 16
NW = NC * NS
B_PER_W = BATCH // NW
FIRE = 16
NITER = B_PER_W // FIRE


CHUNK = 128
NCHUNK = B_PER_W // CHUNK


def _gather_body(table_hbm, idx_hbm, out_hbm, idx_v, rows_v, sem):
    wid = lax.axis_index("s") * NC + lax.axis_index("c")
    base = wid * B_PER_W
    pltpu.sync_copy(idx_hbm.at[wid], idx_v)
    copies = []
    for j in range(NCHUNK):
        copies.append(
            pltpu.async_copy(
                table_hbm.at[idx_v.at[j]],
                rows_v.at[pl.ds(j * CHUNK, CHUNK)],
                sem,
            )
        )
    for c in copies:
        c.wait()
    pltpu.sync_copy(rows_v, out_hbm.at[pl.ds(base, B_PER_W)])


_sc_gather = pl.kernel(
    _gather_body,
    mesh=plsc.VectorSubcoreMesh(core_axis_name="c", subcore_axis_name="s"),
    out_type=jax.ShapeDtypeStruct((BATCH, EMB), jnp.float32),
    scratch_types=[
        pltpu.VMEM((NCHUNK, CHUNK), jnp.int32),
        pltpu.VMEM((B_PER_W, EMB), jnp.float32),
        pltpu.SemaphoreType.DMA,
    ],
    compiler_params=pltpu.CompilerParams(use_tc_tiling_on_sc=False),
)


BM = 1024


def _proj_body(feat_ref, w_ref, b_ref, gath_ref, out_ref):
    left = lax.dot_general(
        feat_ref[...],
        w_ref[...],
        (((1,), (1,)), ((), ())),
        preferred_element_type=jnp.float32,
    )
    left = left + b_ref[...]
    out_ref[...] = jnp.concatenate([left, gath_ref[...]], axis=-1)


def _projector(feat, W, b2, gathered):
    return pl.pallas_call(
        _proj_body,
        grid=(BATCH // BM,),
        in_specs=[
            pl.BlockSpec((BM, FEAT), lambda i: (i, 0)),
            pl.BlockSpec((HID, FEAT), lambda i: (0, 0)),
            pl.BlockSpec((1, HID), lambda i: (0, 0)),
            pl.BlockSpec((BM, EMB), lambda i: (i, 0)),
        ],
        out_specs=pl.BlockSpec((BM, HID + EMB), lambda i: (i, 0)),
        out_shape=jax.ShapeDtypeStruct((BATCH, HID + EMB), jnp.float32),
    )(feat, W, b2, gathered)


def kernel(feat, id, W, b, table):
    ids = id.astype(jnp.int32).reshape(NW, NCHUNK, CHUNK)
    table_lin = lax.optimization_barrier(table.reshape(-1)).reshape(
        table.shape
    )
    gathered = _sc_gather(table_lin, ids)
    return _projector(feat, W, b.reshape(1, HID), gathered)
